# baseline (speedup 1.0000x reference)
import functools

import jax
import jax.numpy as jnp
from jax import lax
from jax.experimental import pallas as pl
from jax.experimental.pallas import tpu as pltpu
from jax.experimental.pallas import tpu_sc as plsc

NC = 2
NS = 16
L = 16
NW = NC * NS
K = 32
C = 4
KC = K * C
CH = 128


def _mf_body(tx, itw, buw, biw, bias, utw, uaw, out,
             txv, uidv, iidv, riidv, offv, uav, utv, itv, buv, biv, outv,
             biasv, sem):
    b = out.shape[0]
    bpw = b // NW
    nchunk = bpw // CH
    wid = lax.axis_index("s") * NC + lax.axis_index("c")
    base = wid * bpw

    pltpu.sync_copy(bias, biasv.at[pl.ds(0, 1)])
    b0 = biasv[...][0]
    lane = lax.iota(jnp.int32, L)

    for ci in range(nchunk):
        cbase = base + ci * CH
        pltpu.sync_copy(tx.at[pl.ds(2 * cbase, 2 * CH)], txv)

        def extract(j, _):
            e2 = 2 * (j * L) + 2 * lane
            u = plsc.load_gather(txv, [e2])
            i = plsc.load_gather(txv, [e2 + 1])
            uidv[pl.ds(j * L, L)] = u
            riidv[pl.ds(j * L, L)] = i
            iidv[pl.ds(j * L, L)] = i >> 2
            offv[pl.ds(j * L, L)] = (i & 3) << 5
            return 0
        lax.fori_loop(0, CH // L, extract, 0)

        cps = [pltpu.async_copy(uaw.at[uidv], uav, sem),
               pltpu.async_copy(utw.at[uidv], utv, sem),
               pltpu.async_copy(itw.at[iidv], itv, sem),
               pltpu.async_copy(buw.at[uidv], buv, sem),
               pltpu.async_copy(biw.at[riidv], biv, sem)]
        for cp in cps:
            cp.wait()

        def group(g, _):
            e = g * L + lane
            off = plsc.load_gather(offv, [e])

            def kbody(k, carry):
                z0, z1, z2, z3, w0, w1, w2, w3 = carry
                kk = jnp.full((L,), k, jnp.int32)
                vi = plsc.load_gather(itv, [e, off + kk])
                c0 = 4 * k
                f0 = jnp.full((L,), c0, jnp.int32)
                f1 = jnp.full((L,), c0 + 1, jnp.int32)
                f2 = jnp.full((L,), c0 + 2, jnp.int32)
                f3 = jnp.full((L,), c0 + 3, jnp.int32)
                uts = (plsc.load_gather(utv, [e, f0])
                       + plsc.load_gather(utv, [e, f1])
                       + plsc.load_gather(utv, [e, f2])
                       + plsc.load_gather(utv, [e, f3]))
                p = vi * uts
                a0 = jnp.exp(plsc.load_gather(uav, [e, f0]) * vi)
                a1 = jnp.exp(plsc.load_gather(uav, [e, f1]) * vi)
                a2 = jnp.exp(plsc.load_gather(uav, [e, f2]) * vi)
                a3 = jnp.exp(plsc.load_gather(uav, [e, f3]) * vi)
                return (z0 + a0, z1 + a1, z2 + a2, z3 + a3,
                        w0 + p * a0, w1 + p * a1, w2 + p * a2, w3 + p * a3)

            zf = jnp.zeros((L,), jnp.float32)
            z0, z1, z2, z3, w0, w1, w2, w3 = lax.fori_loop(
                0, K, kbody, (zf, zf, zf, zf, zf, zf, zf, zf))
            dot = w0 / z0 + w1 / z1 + w2 / z2 + w3 / z3
            bu = buv[pl.ds(g * L, L)]
            bi_ = biv[pl.ds(g * L, L)]
            outv[pl.ds(g * L, L)] = dot + b0 + bu + bi_
            return 0
        lax.fori_loop(0, CH // L, group, 0)

        pltpu.sync_copy(outv, out.at[pl.ds(cbase, CH)])


def kernel(train_x, item_w, bias_user_w, bias_item_w, bias, user_taste, user_attnd):
    b = train_x.shape[0]
    assert b % (NW * CH) == 0
    tx = train_x.astype(jnp.int32).reshape(-1)
    ut2 = user_taste.reshape(user_taste.shape[0], KC)
    ua2 = user_attnd.reshape(user_attnd.shape[0], KC)
    it2 = item_w.reshape(-1, KC)
    bu1 = bias_user_w.reshape(-1)
    bi1 = bias_item_w.reshape(-1)
    mesh = plsc.VectorSubcoreMesh(core_axis_name="c", subcore_axis_name="s")
    kfn = pl.kernel(
        _mf_body,
        mesh=mesh,
        compiler_params=pltpu.CompilerParams(needs_layout_passes=False),
        out_type=jax.ShapeDtypeStruct((b,), jnp.float32),
        scratch_types=[
            pltpu.VMEM((2 * CH,), jnp.int32),
            pltpu.VMEM((CH,), jnp.int32),
            pltpu.VMEM((CH,), jnp.int32),
            pltpu.VMEM((CH,), jnp.int32),
            pltpu.VMEM((CH,), jnp.int32),
            pltpu.VMEM((CH, KC), jnp.float32),
            pltpu.VMEM((CH, KC), jnp.float32),
            pltpu.VMEM((CH, KC), jnp.float32),
            pltpu.VMEM((CH,), jnp.float32),
            pltpu.VMEM((CH,), jnp.float32),
            pltpu.VMEM((CH,), jnp.float32),
            pltpu.VMEM((L,), jnp.float32),
            pltpu.SemaphoreType.DMA,
        ],
    )
    return kfn(tx, it2, bu1, bi1, bias, ut2, ua2)


# speedup vs baseline: 1.1197x; 1.1197x over previous
import functools

import jax
import jax.numpy as jnp
from jax import lax
from jax.experimental import pallas as pl
from jax.experimental.pallas import tpu as pltpu
from jax.experimental.pallas import tpu_sc as plsc

NC = 2
NS = 16
L = 16
NW = NC * NS
K = 32
C = 4
KC = K * C
CH = 128


def _mf_body(tx, itw, buw, biw, bias, utw, uaw, out,
             txv, uidv, iidv, uav, utv, itv, buv, biv, outv, biasv,
             sem0, sem1):
    b = out.shape[0]
    bpw = b // NW
    nchunk = bpw // CH
    wid = lax.axis_index("s") * NC + lax.axis_index("c")
    base = wid * bpw
    sems = (sem0, sem1)

    pltpu.sync_copy(bias, biasv.at[pl.ds(0, 1)])
    b0 = biasv[...][0]
    zeros16 = jnp.zeros((L,), jnp.int32)
    ones16 = jnp.ones((L,), jnp.int32)
    lane = lax.iota(jnp.int32, L)

    def fire(ci, s):
        cbase = base + ci * CH
        ss = jnp.full((L,), s, jnp.int32)
        pltpu.sync_copy(tx.at[pl.ds(cbase, CH)], txv.at[s])

        def extract(j, _):
            e = j * L + lane
            uidv[s, pl.ds(j * L, L)] = plsc.load_gather(txv, [ss, e, zeros16])
            iidv[s, pl.ds(j * L, L)] = plsc.load_gather(txv, [ss, e, ones16])
            return 0
        lax.fori_loop(0, CH // L, extract, 0)
        return [pltpu.async_copy(uaw.at[uidv.at[s]], uav.at[s], sems[s]),
                pltpu.async_copy(utw.at[uidv.at[s]], utv.at[s], sems[s]),
                pltpu.async_copy(itw.at[iidv.at[s]], itv.at[s], sems[s]),
                pltpu.async_copy(buw.at[uidv.at[s]], buv.at[s], sems[s]),
                pltpu.async_copy(biw.at[iidv.at[s]], biv.at[s], sems[s])]

    def compute(ci, s):
        cbase = base + ci * CH
        ss = jnp.full((L,), s, jnp.int32)

        def group(g, _):
            e = g * L + lane
            zf = jnp.zeros((L,), jnp.float32)

            @plsc.parallel_loop(0, K, unroll=4,
                                carry=(zf, zf, zf, zf, zf, zf, zf, zf))
            def acc(k, carry):
                z0, z1, z2, z3, w0, w1, w2, w3 = carry
                kk = jnp.full((L,), k, jnp.int32)
                vi = plsc.load_gather(itv, [ss, e, kk])
                c0 = 4 * k
                f0 = jnp.full((L,), c0, jnp.int32)
                f1 = jnp.full((L,), c0 + 1, jnp.int32)
                f2 = jnp.full((L,), c0 + 2, jnp.int32)
                f3 = jnp.full((L,), c0 + 3, jnp.int32)
                uts = (plsc.load_gather(utv, [ss, e, f0])
                       + plsc.load_gather(utv, [ss, e, f1])
                       + plsc.load_gather(utv, [ss, e, f2])
                       + plsc.load_gather(utv, [ss, e, f3]))
                p = vi * uts
                a0 = jnp.exp(plsc.load_gather(uav, [ss, e, f0]) * vi)
                a1 = jnp.exp(plsc.load_gather(uav, [ss, e, f1]) * vi)
                a2 = jnp.exp(plsc.load_gather(uav, [ss, e, f2]) * vi)
                a3 = jnp.exp(plsc.load_gather(uav, [ss, e, f3]) * vi)
                return (z0 + a0, z1 + a1, z2 + a2, z3 + a3,
                        w0 + p * a0, w1 + p * a1, w2 + p * a2, w3 + p * a3)

            z0, z1, z2, z3, w0, w1, w2, w3 = acc
            dot = w0 / z0 + w1 / z1 + w2 / z2 + w3 / z3
            bu = buv[s, pl.ds(g * L, L)]
            bi_ = biv[s, pl.ds(g * L, L)]
            outv[pl.ds(g * L, L)] = dot + b0 + bu + bi_
            return 0
        lax.fori_loop(0, CH // L, group, 0)
        pltpu.sync_copy(outv, out.at[pl.ds(cbase, CH)])

    pend = {0: fire(0, 0)}
    for ci in range(nchunk):
        s = ci % 2
        if ci + 1 < nchunk:
            pend[ci + 1] = fire(ci + 1, 1 - s)
        for cp in pend.pop(ci):
            cp.wait()
        compute(ci, s)


def kernel(train_x, item_w, bias_user_w, bias_item_w, bias, user_taste, user_attnd):
    b = train_x.shape[0]
    assert b % (NW * CH) == 0
    tx = train_x.astype(jnp.int32)
    ut2 = user_taste.reshape(user_taste.shape[0], KC)
    ua2 = user_attnd.reshape(user_attnd.shape[0], KC)
    bu1 = bias_user_w.reshape(-1)
    bi1 = bias_item_w.reshape(-1)
    mesh = plsc.VectorSubcoreMesh(core_axis_name="c", subcore_axis_name="s")
    kfn = pl.kernel(
        _mf_body,
        mesh=mesh,
        compiler_params=pltpu.CompilerParams(
            needs_layout_passes=False, use_tc_tiling_on_sc=False),
        out_type=jax.ShapeDtypeStruct((b,), jnp.float32),
        scratch_types=[
            pltpu.VMEM((2, CH, 2), jnp.int32),
            pltpu.VMEM((2, CH), jnp.int32),
            pltpu.VMEM((2, CH), jnp.int32),
            pltpu.VMEM((2, CH, KC), jnp.float32),
            pltpu.VMEM((2, CH, KC), jnp.float32),
            pltpu.VMEM((2, CH, K), jnp.float32),
            pltpu.VMEM((2, CH), jnp.float32),
            pltpu.VMEM((2, CH), jnp.float32),
            pltpu.VMEM((CH,), jnp.float32),
            pltpu.VMEM((L,), jnp.float32),
            pltpu.SemaphoreType.DMA,
            pltpu.SemaphoreType.DMA,
        ],
    )
    return kfn(tx, item_w, bu1, bi1, bias, ut2, ua2)


# speedup vs baseline: 1.1202x; 1.0005x over previous
import jax
import jax.numpy as jnp
from jax import lax
from jax.experimental import pallas as pl
from jax.experimental.pallas import tpu as pltpu
from jax.experimental.pallas import tpu_sc as plsc

NC = 2
NS = 16
L = 16
NW = NC * NS
K = 32
C = 4
KC = K * C
CH = 128


def _mf_body(tx, itw, buw, biw, bias, utw, uaw, out,
             txv, uidv, iidv, uav, utv, itv, buv, biv, outv, biasv,
             sem0, sem1):
    b = out.shape[0]
    bpw = b // NW
    nchunk = bpw // CH
    wid = lax.axis_index("s") * NC + lax.axis_index("c")
    base = wid * bpw
    sems = (sem0, sem1)

    pltpu.sync_copy(bias, biasv.at[pl.ds(0, 1)])
    b0 = biasv[...][0]
    zeros16 = jnp.zeros((L,), jnp.int32)
    ones16 = jnp.ones((L,), jnp.int32)
    lane = lax.iota(jnp.int32, L)

    def fire(ci, s):
        cbase = base + ci * CH
        ss = jnp.full((L,), s, jnp.int32)
        pltpu.sync_copy(tx.at[pl.ds(cbase, CH)], txv.at[s])

        def extract(j, _):
            e = j * L + lane
            uidv[s, pl.ds(j * L, L)] = plsc.load_gather(txv, [ss, e, zeros16])
            iidv[s, pl.ds(j * L, L)] = plsc.load_gather(txv, [ss, e, ones16])
            return 0
        lax.fori_loop(0, CH // L, extract, 0)
        return [pltpu.async_copy(uaw.at[uidv.at[s]], uav.at[s], sems[s]),
                pltpu.async_copy(utw.at[uidv.at[s]], utv.at[s], sems[s]),
                pltpu.async_copy(itw.at[iidv.at[s]], itv.at[s], sems[s]),
                pltpu.async_copy(buw.at[uidv.at[s]], buv.at[s], sems[s]),
                pltpu.async_copy(biw.at[iidv.at[s]], biv.at[s], sems[s])]

    def compute(ci, s):
        cbase = base + ci * CH
        ss = jnp.full((L,), s, jnp.int32)

        def group(g, _):
            e = g * L + lane
            zf = jnp.zeros((L,), jnp.float32)

            @plsc.parallel_loop(0, K, unroll=4,
                                carry=(zf, zf, zf, zf, zf, zf, zf, zf))
            def acc(k, carry):
                z0, z1, z2, z3, w0, w1, w2, w3 = carry
                kk = jnp.full((L,), k, jnp.int32)
                vi = plsc.load_gather(itv, [ss, e, kk])
                c0 = 4 * k
                f0 = jnp.full((L,), c0, jnp.int32)
                f1 = jnp.full((L,), c0 + 1, jnp.int32)
                f2 = jnp.full((L,), c0 + 2, jnp.int32)
                f3 = jnp.full((L,), c0 + 3, jnp.int32)
                uts = (plsc.load_gather(utv, [ss, e, f0])
                       + plsc.load_gather(utv, [ss, e, f1])
                       + plsc.load_gather(utv, [ss, e, f2])
                       + plsc.load_gather(utv, [ss, e, f3]))
                p = vi * uts
                a0 = jnp.exp(plsc.load_gather(uav, [ss, e, f0]) * vi)
                a1 = jnp.exp(plsc.load_gather(uav, [ss, e, f1]) * vi)
                a2 = jnp.exp(plsc.load_gather(uav, [ss, e, f2]) * vi)
                a3 = jnp.exp(plsc.load_gather(uav, [ss, e, f3]) * vi)
                return (z0 + a0, z1 + a1, z2 + a2, z3 + a3,
                        w0 + p * a0, w1 + p * a1, w2 + p * a2, w3 + p * a3)

            z0, z1, z2, z3, w0, w1, w2, w3 = acc
            dot = w0 / z0 + w1 / z1 + w2 / z2 + w3 / z3
            bu = buv[s, pl.ds(g * L, L)]
            bi_ = biv[s, pl.ds(g * L, L)]
            outv[pl.ds(g * L, L)] = dot + b0 + bu + bi_
            return 0
        lax.fori_loop(0, CH // L, group, 0)
        pltpu.sync_copy(outv, out.at[pl.ds(cbase, CH)])

    pend = {0: fire(0, 0)}
    for ci in range(nchunk):
        s = ci % 2
        if ci + 1 < nchunk:
            pend[ci + 1] = fire(ci + 1, 1 - s)
        for cp in pend.pop(ci):
            cp.wait()
        compute(ci, s)


def kernel(train_x, item_w, bias_user_w, bias_item_w, bias, user_taste, user_attnd):
    b = train_x.shape[0]
    assert b % (NW * CH) == 0
    tx = train_x.astype(jnp.int32)
    ut2 = user_taste.reshape(user_taste.shape[0], KC)
    ua2 = user_attnd.reshape(user_attnd.shape[0], KC)
    bu1 = bias_user_w.reshape(-1)
    bi1 = bias_item_w.reshape(-1)
    mesh = plsc.VectorSubcoreMesh(core_axis_name="c", subcore_axis_name="s")
    kfn = pl.kernel(
        _mf_body,
        mesh=mesh,
        compiler_params=pltpu.CompilerParams(
            needs_layout_passes=False, use_tc_tiling_on_sc=False),
        out_type=jax.ShapeDtypeStruct((b,), jnp.float32),
        scratch_types=[
            pltpu.VMEM((2, CH, 2), jnp.int32),
            pltpu.VMEM((2, CH), jnp.int32),
            pltpu.VMEM((2, CH), jnp.int32),
            pltpu.VMEM((2, CH, KC), jnp.float32),
            pltpu.VMEM((2, CH, KC), jnp.float32),
            pltpu.VMEM((2, CH, K), jnp.float32),
            pltpu.VMEM((2, CH), jnp.float32),
            pltpu.VMEM((2, CH), jnp.float32),
            pltpu.VMEM((CH,), jnp.float32),
            pltpu.VMEM((L,), jnp.float32),
            pltpu.SemaphoreType.DMA,
            pltpu.SemaphoreType.DMA,
        ],
    )
    return kfn(tx, item_w, bu1, bi1, bias, ut2, ua2)
